# baseline (speedup 1.0000x reference)
import functools
import math

import jax
import jax.numpy as jnp
from jax import lax
from jax.experimental import pallas as pl
from jax.experimental.pallas import tpu as pltpu
from jax.experimental.pallas import tpu_sc as plsc

VOCAB = 1000000
D = 64
S = 200
B = 4096

NC = 2
NS = 16
NW = NC * NS

T = B * S
PER_W = T // NW
CH = 128
NCH = PER_W // CH
LANES = 16
SCALE = math.sqrt(D)


def _pos_encoding():
    position = jnp.arange(0, S, dtype=jnp.float32)[:, None]
    div_term = jnp.exp(
        jnp.arange(0, D, 2, dtype=jnp.float32) * -(math.log(10000.0) / D)
    )
    pe = jnp.zeros((S, D), dtype=jnp.float32)
    pe = pe.at[:, 0::2].set(jnp.sin(position * div_term))
    pe = pe.at[:, 1::2].set(jnp.cos(position * div_term))
    return pe


def _make_sc_kernel():
    mesh = plsc.VectorSubcoreMesh(core_axis_name="c", subcore_axis_name="s")

    @functools.partial(
        pl.kernel,
        out_type=jax.ShapeDtypeStruct((T, D), jnp.float32),
        mesh=mesh,
        scratch_types=[
            pltpu.VMEM((NCH, CH), jnp.int32),
            pltpu.VMEM((2 * S, D), jnp.float32),
            pltpu.VMEM((CH, D), jnp.float32),
            pltpu.SemaphoreType.DMA,
        ],
        compiler_params=pltpu.CompilerParams(use_tc_tiling_on_sc=False),
    )
    def sc_body(table_hbm, idx_hbm, pe_hbm, out_hbm, idx_v, pe_v, rows_v, gsem):
        wid = lax.axis_index("s") * NC + lax.axis_index("c")
        base = wid * PER_W
        pltpu.sync_copy(idx_hbm.at[wid], idx_v)
        pltpu.sync_copy(pe_hbm, pe_v)

        def chunk(c, carry):
            pltpu.async_copy(table_hbm.at[idx_v.at[c]], rows_v, gsem).wait()
            pe_off = lax.rem(c * CH, S)

            def row(i, carry2):
                for j in range(D // LANES):
                    sl = pl.ds(j * LANES, LANES)
                    rows_v[i, sl] = rows_v[i, sl] * SCALE + pe_v[pe_off + i, sl]
                return carry2

            lax.fori_loop(0, CH, row, 0)
            pltpu.sync_copy(rows_v, out_hbm.at[pl.ds(base + c * CH, CH)])
            return carry

        lax.fori_loop(0, NCH, chunk, 0)

    return sc_body


_sc_kernel = _make_sc_kernel()


def kernel(input_token_ids, token_embedding):
    idx = input_token_ids.astype(jnp.int32).reshape(NW, NCH, CH)
    pe = _pos_encoding()
    pe2 = jnp.concatenate([pe, pe], axis=0)
    out = _sc_kernel(token_embedding, idx, pe2)
    return out.reshape(B, S, D)


# speedup vs baseline: 1.5369x; 1.5369x over previous
import functools
import math

import jax
import jax.numpy as jnp
from jax import lax
from jax.experimental import pallas as pl
from jax.experimental.pallas import tpu as pltpu
from jax.experimental.pallas import tpu_sc as plsc

VOCAB = 1000000
D = 64
S = 200
B = 4096

NC = 2
NS = 16
NW = NC * NS

T = B * S
PER_W = T // NW
CH = 128
NCH = PER_W // CH
NBUF = 4
LANES = 16
SCALE = math.sqrt(D)


def _pos_encoding():
    position = jnp.arange(0, S, dtype=jnp.float32)[:, None]
    div_term = jnp.exp(
        jnp.arange(0, D, 2, dtype=jnp.float32) * -(math.log(10000.0) / D)
    )
    pe = jnp.zeros((S, D), dtype=jnp.float32)
    pe = pe.at[:, 0::2].set(jnp.sin(position * div_term))
    pe = pe.at[:, 1::2].set(jnp.cos(position * div_term))
    return pe


def _make_sc_kernel():
    mesh = plsc.VectorSubcoreMesh(core_axis_name="c", subcore_axis_name="s")

    @functools.partial(
        pl.kernel,
        out_type=jax.ShapeDtypeStruct((T, D), jnp.float32),
        mesh=mesh,
        scratch_types=[
            pltpu.VMEM((NCH, CH), jnp.int32),
            pltpu.VMEM((2 * S, D), jnp.float32),
            [pltpu.VMEM((CH, D), jnp.float32)] * NBUF,
            [pltpu.SemaphoreType.DMA] * NBUF,
            [pltpu.SemaphoreType.DMA] * NBUF,
        ],
        compiler_params=pltpu.CompilerParams(use_tc_tiling_on_sc=False),
    )
    def sc_body(table_hbm, idx_hbm, pe_hbm, out_hbm, idx_v, pe_v, rows, gsem, osem):
        wid = lax.axis_index("s") * NC + lax.axis_index("c")
        base = wid * PER_W
        pltpu.sync_copy(idx_hbm.at[wid], idx_v)
        pltpu.sync_copy(pe_hbm, pe_v)

        def gather(k, b):
            return pltpu.make_async_copy(
                table_hbm.at[idx_v.at[k]], rows[b], gsem[b]
            )

        def writeout(k, b):
            return pltpu.make_async_copy(
                rows[b], out_hbm.at[pl.ds(base + k * CH, CH)], osem[b]
            )

        gather(0, 0).start()
        gather(1, 1).start()

        @pl.loop(0, NCH, step=NBUF)
        def _(c):
            for b in range(NBUF):
                k = c + b
                b2 = (b + 2) % NBUF

                @pl.when(k + 2 < NCH)
                def _():
                    @pl.when(k >= 2)
                    def _():
                        writeout(k - 2, b2).wait()

                    gather(k + 2, b2).start()

                gather(k, b).wait()

                pe_off = lax.rem(k * CH, S)
                rows_b = rows[b]

                @plsc.parallel_loop(0, CH, unroll=4)
                def _(i):
                    for j in range(D // LANES):
                        sl = pl.ds(j * LANES, LANES)
                        rows_b[i, sl] = rows_b[i, sl] * SCALE + pe_v[pe_off + i, sl]

                writeout(k, b).start()

        for b in range(NBUF):
            writeout(NCH - NBUF + b, b).wait()

    return sc_body


_sc_kernel = _make_sc_kernel()


def kernel(input_token_ids, token_embedding):
    idx = input_token_ids.astype(jnp.int32).reshape(NW, NCH, CH)
    pe = _pos_encoding()
    pe2 = jnp.concatenate([pe, pe], axis=0)
    out = _sc_kernel(token_embedding, idx, pe2)
    return out.reshape(B, S, D)
